# Initial kernel scaffold; baseline (speedup 1.0000x reference)
import jax
import jax.numpy as jnp
from jax.experimental import pallas as pl


def kernel(conv1_w, conv1_b, conv2_w, conv2_b, lin1_w, lin1_b, lin2_w, lin2_b, x_nchw, xc):
    raise NotImplementedError("write your pallas kernel here")


# speedup vs baseline: 1.1290x; 1.1290x over previous
import jax
import jax.numpy as jnp
from jax.experimental import pallas as pl
from jax.experimental.pallas import tpu as pltpu


def _conv_stack_kernel(x_ref, w1_ref, b1_ref, w2_ref, b2_ref, o_ref):
    Bb = x_ref.shape[0]
    H = x_ref.shape[1]
    W = x_ref.shape[2]
    C1 = b1_ref.shape[1]
    C2 = b2_ref.shape[1]
    Hp, Wp = H // 2, W // 2
    Hq, Wq = Hp // 2, Wp // 2

    xp = jnp.pad(x_ref[...], ((0, 0), (1, 1), (1, 1), (0, 0)))
    L1 = Bb * (H + 2) * (W + 2)
    P = jnp.dot(xp.reshape(L1, xp.shape[-1]), w1_ref[...],
                preferred_element_type=jnp.float32)
    P = jnp.pad(P, ((0, 2 * (W + 2) + 2), (0, 0)))
    acc = jnp.zeros((L1, C1), jnp.float32)
    for dy in range(3):
        for dx in range(3):
            off = dy * (W + 2) + dx
            t = dy * 3 + dx
            acc = acc + P[off:off + L1, t * C1:(t + 1) * C1]
    acc = acc.reshape(Bb, H + 2, W + 2, C1)[:, :H, :W, :]
    y = jnp.maximum(acc + b1_ref[...], 0.0)

    y = y.reshape(Bb * H * Wp, 2, C1)
    y = jnp.maximum(y[:, 0, :], y[:, 1, :])
    y = y.reshape(Bb * Hp, 2, Wp, C1)
    y = jnp.maximum(y[:, 0], y[:, 1])
    y1 = y.reshape(Bb, Hp, Wp, C1).astype(jnp.bfloat16)

    yp = jnp.pad(y1, ((0, 0), (1, 1), (1, 1), (0, 0)))
    L2 = Bb * (Hp + 2) * (Wp + 2)
    P2 = jnp.dot(yp.reshape(L2, C1), w2_ref[...],
                 preferred_element_type=jnp.float32)
    P2 = jnp.pad(P2, ((0, 2 * (Wp + 2) + 2), (0, 0)))
    acc2 = jnp.zeros((L2, C2), jnp.float32)
    for dy in range(3):
        for dx in range(3):
            off = dy * (Wp + 2) + dx
            t = dy * 3 + dx
            acc2 = acc2 + P2[off:off + L2, t * C2:(t + 1) * C2]
    acc2 = acc2.reshape(Bb, Hp + 2, Wp + 2, C2)[:, :Hp, :Wp, :]
    z = jnp.maximum(acc2 + b2_ref[...], 0.0)

    z = z.reshape(Bb * Hp * Wq, 2, C2)
    z = jnp.maximum(z[:, 0, :], z[:, 1, :])
    z = z.reshape(Bb * Hq, 2, Wq, C2)
    z = jnp.maximum(z[:, 0], z[:, 1])

    o_ref[...] = z.reshape(Bb, Hq, Wq, C2).astype(jnp.bfloat16)


def _conv_stack(x, w1_all, b1, w2_all, b2, block_b):
    B, H, W, Cin = x.shape
    C1 = b1.shape[1]
    C2 = b2.shape[1]
    Hq, Wq = H // 4, W // 4

    return pl.pallas_call(
        _conv_stack_kernel,
        out_shape=jax.ShapeDtypeStruct((B, Hq, Wq, C2), jnp.bfloat16),
        grid=(B // block_b,),
        in_specs=[
            pl.BlockSpec((block_b, H, W, Cin), lambda n: (n, 0, 0, 0)),
            pl.BlockSpec((Cin, 9 * C1), lambda n: (0, 0)),
            pl.BlockSpec((1, C1), lambda n: (0, 0)),
            pl.BlockSpec((C1, 9 * C2), lambda n: (0, 0)),
            pl.BlockSpec((1, C2), lambda n: (0, 0)),
        ],
        out_specs=pl.BlockSpec((block_b, Hq, Wq, C2), lambda n: (n, 0, 0, 0)),
        compiler_params=pltpu.CompilerParams(
            dimension_semantics=("parallel",),
            vmem_limit_bytes=63 * 1024 * 1024),
        cost_estimate=pl.CostEstimate(
            flops=2 * B * (H * W * 9 * Cin * C1 + (H // 2) * (W // 2) * 9 * C1 * C2),
            transcendentals=0,
            bytes_accessed=2 * x.size + 2 * B * Hq * Wq * C2),
    )(x, w1_all, b1, w2_all, b2)


def _mlp_kernel(x_ref, w1_ref, b1_ref, w2_ref, b2_ref, o_ref):
    h = jnp.dot(x_ref[...], w1_ref[...], preferred_element_type=jnp.float32)
    h = jnp.maximum(h + b1_ref[...], 0.0).astype(jnp.bfloat16)
    o = jnp.dot(h, w2_ref[...], preferred_element_type=jnp.float32)
    o_ref[...] = jnp.maximum(o + b2_ref[...], 0.0)


def _mlp_head(x, w1, b1, w2, b2, block_b):
    B, Din = x.shape
    H1 = w1.shape[1]
    H2 = w2.shape[1]

    return pl.pallas_call(
        _mlp_kernel,
        out_shape=jax.ShapeDtypeStruct((B, H2), jnp.float32),
        grid=(B // block_b,),
        in_specs=[
            pl.BlockSpec((block_b, Din), lambda n: (n, 0)),
            pl.BlockSpec((Din, H1), lambda n: (0, 0)),
            pl.BlockSpec((1, H1), lambda n: (0, 0)),
            pl.BlockSpec((H1, H2), lambda n: (0, 0)),
            pl.BlockSpec((1, H2), lambda n: (0, 0)),
        ],
        out_specs=pl.BlockSpec((block_b, H2), lambda n: (n, 0)),
        compiler_params=pltpu.CompilerParams(
            dimension_semantics=("parallel",),
            vmem_limit_bytes=64 * 1024 * 1024),
        cost_estimate=pl.CostEstimate(
            flops=2 * B * (Din * H1 + H1 * H2),
            transcendentals=0,
            bytes_accessed=2 * (x.size + w1.size + w2.size) + 4 * B * H2),
    )(x, w1, b1, w2, b2)


def kernel(conv1_w, conv1_b, conv2_w, conv2_b, lin1_w, lin1_b,
           lin2_w, lin2_b, x_nchw, xc):
    del xc
    B = x_nchw.shape[0]

    x = jnp.transpose(x_nchw, (0, 2, 3, 1)).astype(jnp.bfloat16)

    c1 = conv1_w.shape[2]
    c2 = conv2_w.shape[2]
    w1_all = jnp.transpose(conv1_w, (1, 0, 2)).reshape(conv1_w.shape[1], 9 * c1)
    w2_all = jnp.transpose(conv2_w, (1, 0, 2)).reshape(conv2_w.shape[1], 9 * c2)

    y2 = _conv_stack(x, w1_all.astype(jnp.bfloat16), conv1_b,
                     w2_all.astype(jnp.bfloat16), conv2_b, block_b=2)

    flat = y2.reshape(B, y2.shape[1] * y2.shape[2] * y2.shape[3])
    return _mlp_head(flat, lin1_w.astype(jnp.bfloat16), lin1_b,
                     lin2_w.astype(jnp.bfloat16), lin2_b, block_b=128)


# speedup vs baseline: 1.4917x; 1.3213x over previous
import jax
import jax.numpy as jnp
from jax.experimental import pallas as pl
from jax.experimental.pallas import tpu as pltpu


def _conv_stack_kernel(x_ref, w1_ref, b1_ref, w2_ref, b2_ref, o_ref):
    Bb = x_ref.shape[0]
    H = x_ref.shape[2]
    W = x_ref.shape[3]
    C1 = b1_ref.shape[1]
    C2 = b2_ref.shape[1]
    Hp, Wp = H // 2, W // 2
    Hq, Wq = Hp // 2, Wp // 2

    xh = jnp.transpose(x_ref[...].astype(jnp.bfloat16), (0, 2, 3, 1))
    xp = jnp.pad(xh, ((0, 0), (1, 1), (1, 1), (0, 0)))
    L1 = Bb * (H + 2) * (W + 2)
    P = jnp.dot(xp.reshape(L1, xp.shape[-1]), w1_ref[...],
                preferred_element_type=jnp.float32)
    P = jnp.pad(P, ((0, 2 * (W + 2) + 2), (0, 0)))
    acc = jnp.zeros((L1, C1), jnp.float32)
    for dy in range(3):
        for dx in range(3):
            off = dy * (W + 2) + dx
            t = dy * 3 + dx
            acc = acc + P[off:off + L1, t * C1:(t + 1) * C1]
    acc = acc.reshape(Bb, H + 2, W + 2, C1)[:, :H, :W, :]
    y = jnp.maximum(acc + b1_ref[...], 0.0)

    y = y.reshape(Bb * H * Wp, 2, C1)
    y = jnp.maximum(y[:, 0, :], y[:, 1, :])
    y = y.reshape(Bb * Hp, 2, Wp, C1)
    y = jnp.maximum(y[:, 0], y[:, 1])
    y1 = y.reshape(Bb, Hp, Wp, C1).astype(jnp.bfloat16)

    yp = jnp.pad(y1, ((0, 0), (1, 1), (1, 1), (0, 0)))
    L2 = Bb * (Hp + 2) * (Wp + 2)
    P2 = jnp.dot(yp.reshape(L2, C1), w2_ref[...],
                 preferred_element_type=jnp.float32)
    P2 = jnp.pad(P2, ((0, 2 * (Wp + 2) + 2), (0, 0)))
    acc2 = jnp.zeros((L2, C2), jnp.float32)
    for dy in range(3):
        for dx in range(3):
            off = dy * (Wp + 2) + dx
            t = dy * 3 + dx
            acc2 = acc2 + P2[off:off + L2, t * C2:(t + 1) * C2]
    acc2 = acc2.reshape(Bb, Hp + 2, Wp + 2, C2)[:, :Hp, :Wp, :]
    z = jnp.maximum(acc2 + b2_ref[...], 0.0)

    z = z.reshape(Bb * Hp * Wq, 2, C2)
    z = jnp.maximum(z[:, 0, :], z[:, 1, :])
    z = z.reshape(Bb * Hq, 2, Wq, C2)
    z = jnp.maximum(z[:, 0], z[:, 1])

    o_ref[...] = z.reshape(Bb, Hq, Wq, C2).astype(jnp.bfloat16)


def _conv_stack(x, w1_all, b1, w2_all, b2, block_b):
    B, Cin, H, W = x.shape
    C1 = b1.shape[1]
    C2 = b2.shape[1]
    Hq, Wq = H // 4, W // 4

    return pl.pallas_call(
        _conv_stack_kernel,
        out_shape=jax.ShapeDtypeStruct((B, Hq, Wq, C2), jnp.bfloat16),
        grid=(B // block_b,),
        in_specs=[
            pl.BlockSpec((block_b, Cin, H, W), lambda n: (n, 0, 0, 0)),
            pl.BlockSpec((Cin, 9 * C1), lambda n: (0, 0)),
            pl.BlockSpec((1, C1), lambda n: (0, 0)),
            pl.BlockSpec((C1, 9 * C2), lambda n: (0, 0)),
            pl.BlockSpec((1, C2), lambda n: (0, 0)),
        ],
        out_specs=pl.BlockSpec((block_b, Hq, Wq, C2), lambda n: (n, 0, 0, 0)),
        compiler_params=pltpu.CompilerParams(
            dimension_semantics=("parallel",),
            vmem_limit_bytes=63 * 1024 * 1024),
        cost_estimate=pl.CostEstimate(
            flops=2 * B * (H * W * 9 * Cin * C1 + (H // 2) * (W // 2) * 9 * C1 * C2),
            transcendentals=0,
            bytes_accessed=2 * x.size + 2 * B * Hq * Wq * C2),
    )(x, w1_all, b1, w2_all, b2)


def _mlp_kernel(x_ref, w1_ref, b1_ref, w2_ref, b2_ref, o_ref):
    h = jnp.dot(x_ref[...], w1_ref[...], preferred_element_type=jnp.float32)
    h = jnp.maximum(h + b1_ref[...], 0.0).astype(jnp.bfloat16)
    o = jnp.dot(h, w2_ref[...], preferred_element_type=jnp.float32)
    o_ref[...] = jnp.maximum(o + b2_ref[...], 0.0)


def _mlp_head(x, w1, b1, w2, b2, block_b):
    B, Din = x.shape
    H1 = w1.shape[1]
    H2 = w2.shape[1]

    return pl.pallas_call(
        _mlp_kernel,
        out_shape=jax.ShapeDtypeStruct((B, H2), jnp.float32),
        grid=(B // block_b,),
        in_specs=[
            pl.BlockSpec((block_b, Din), lambda n: (n, 0)),
            pl.BlockSpec((Din, H1), lambda n: (0, 0)),
            pl.BlockSpec((1, H1), lambda n: (0, 0)),
            pl.BlockSpec((H1, H2), lambda n: (0, 0)),
            pl.BlockSpec((1, H2), lambda n: (0, 0)),
        ],
        out_specs=pl.BlockSpec((block_b, H2), lambda n: (n, 0)),
        compiler_params=pltpu.CompilerParams(
            dimension_semantics=("parallel",),
            vmem_limit_bytes=64 * 1024 * 1024),
        cost_estimate=pl.CostEstimate(
            flops=2 * B * (Din * H1 + H1 * H2),
            transcendentals=0,
            bytes_accessed=2 * (x.size + w1.size + w2.size) + 4 * B * H2),
    )(x, w1, b1, w2, b2)


def kernel(conv1_w, conv1_b, conv2_w, conv2_b, lin1_w, lin1_b,
           lin2_w, lin2_b, x_nchw, xc):
    del xc
    B = x_nchw.shape[0]

    c1 = conv1_w.shape[2]
    c2 = conv2_w.shape[2]
    w1_all = jnp.transpose(conv1_w, (1, 0, 2)).reshape(conv1_w.shape[1], 9 * c1)
    w2_all = jnp.transpose(conv2_w, (1, 0, 2)).reshape(conv2_w.shape[1], 9 * c2)

    y2 = _conv_stack(x_nchw, w1_all.astype(jnp.bfloat16), conv1_b,
                     w2_all.astype(jnp.bfloat16), conv2_b, block_b=2)

    flat = y2.reshape(B, y2.shape[1] * y2.shape[2] * y2.shape[3])
    return _mlp_head(flat, lin1_w.astype(jnp.bfloat16), lin1_b,
                     lin2_w.astype(jnp.bfloat16), lin2_b, block_b=128)
